# baseline (speedup 1.0000x reference)
import functools

import jax
import jax.numpy as jnp
from jax import lax
from jax.experimental import pallas as pl
from jax.experimental.pallas import tpu as pltpu
from jax.experimental.pallas import tpu_sc as plsc

_N_NODES = 100000
_N_EDGES = 6400000
_NC = 2
_NS = 16
_NW = _NC * _NS
_B = 1600
_NBLK = _N_EDGES // _B
_NT = _NBLK // _NW
_STEPS = _B // 16
_N_PAD = 100352
_ZSL = _N_PAD // _NS
_CSL = _N_PAD // _NW

_K = 2048
_DMAX = 14.08
_H = _DMAX / _K
_INV_H = _K / _DMAX
_TABW = _K + 16

_KEHALF = 7.199822675975274
_C1 = 1.0 / 144.0
_C2 = 1.0 / 6.0
_CUTOFF = 12.0

_mesh = plsc.VectorSubcoreMesh(
    core_axis_name="c", subcore_axis_name="s", num_cores=_NC, num_subcores=_NS
)


def _m_exact(d):
    t = d * d + 1.0
    ib = plsc.bitcast(t, jnp.int32)
    ib = jnp.int32(0x5F3759DF) - lax.shift_right_arithmetic(ib, 1)
    r = plsc.bitcast(ib, jnp.float32)
    th = 0.5 * t
    r = r * (1.5 - th * r * r)
    r = r * (1.5 - th * r * r)
    r = r * (1.5 - th * r * r)
    ds_ = t * r
    e_shd = r + (ds_ * jnp.float32(_C1) - jnp.float32(_C2))
    e_ord = 1.0 / d + (d * jnp.float32(_C1) - jnp.float32(_C2))
    x = jnp.minimum(d * 0.5, 1.0)
    x2 = x * x
    x3 = x2 * x
    w_on = x3 * ((6.0 * x2 - 15.0 * x) + 10.0)
    mix = e_shd + w_on * (e_ord - e_shd)
    return jnp.where(d <= jnp.float32(_CUTOFF),
                     jnp.float32(_KEHALF) * mix, jnp.float32(0.0))


@functools.partial(
    pl.kernel,
    out_type=(jax.ShapeDtypeStruct((_N_PAD,), jnp.float32),
              jax.ShapeDtypeStruct((_N_PAD,), jnp.float32)),
    mesh=_mesh,
    compiler_params=pltpu.CompilerParams(needs_layout_passes=False),
    scratch_types=[
        pltpu.VMEM((_N_NODES,), jnp.float32),
        pltpu.VMEM((_TABW,), jnp.float32),
        pltpu.VMEM((_TABW,), jnp.float32),
        pltpu.VMEM((_B,), jnp.float32),
        pltpu.VMEM((_B,), jnp.float32),
        pltpu.VMEM((_B,), jnp.float32),
        pltpu.VMEM((_B,), jnp.int32),
        pltpu.VMEM((_B,), jnp.int32),
        pltpu.VMEM((_B,), jnp.int32),
        pltpu.VMEM((_B,), jnp.int32),
        pltpu.VMEM((_B,), jnp.int32),
        pltpu.VMEM((_B,), jnp.int32),
        pltpu.VMEM((_B,), jnp.float32),
        pltpu.VMEM((_B,), jnp.float32),
        pltpu.VMEM((_B,), jnp.float32),
        pltpu.VMEM_SHARED((_N_PAD,), jnp.float32),
        pltpu.SemaphoreType.DMA,
        pltpu.SemaphoreType.DMA,
        pltpu.SemaphoreType.DMA,
        pltpu.SemaphoreType.DMA,
        pltpu.SemaphoreType.DMA,
        pltpu.SemaphoreType.DMA,
    ],
)
def _sc_main(ch_hbm, d_hbm, ii_hbm, ij_hbm, out0_hbm, out1_hbm,
             table_v, mval_v, mslp_v,
             d0, d1, d2, i0, i1, i2, j0, j1, j2, e0, e1, e2,
             accum_sh, si0, si1, si2, ss0, ss1, ss2):
    cid = lax.axis_index("c")
    sid = lax.axis_index("s")
    wid = sid * _NC + cid
    sets = ((d0, i0, j0, e0, si0, ss0),
            (d1, i1, j1, e1, si1, ss1),
            (d2, i2, j2, e2, si2, ss2))

    pltpu.sync_copy(ch_hbm, table_v)

    lanes = lax.iota(jnp.int32, 16)

    def build_vals(k, _):
        node = k * 16 + lanes
        dn = node.astype(jnp.float32) * jnp.float32(_H)
        dn = jnp.maximum(dn, jnp.float32(0.25 * _H))
        mval_v[pl.ds(k * 16, 16)] = _m_exact(dn)
        return 0
    lax.fori_loop(0, _K // 16, build_vals, 0)
    mval_v[pl.ds(_K, 16)] = jnp.zeros((16,), jnp.float32)

    def build_slopes(k, _):
        m0 = mval_v[pl.ds(k * 16, 16)]
        m1 = mval_v[pl.ds(k * 16 + 1, 16)]
        mslp_v[pl.ds(k * 16, 16)] = m1 - m0
        return 0
    lax.fori_loop(0, _K // 16, build_slopes, 0)

    def zstep(k, _):
        e0[pl.ds(k * 16, 16)] = jnp.zeros((16,), jnp.float32)
        return 0
    lax.fori_loop(0, _STEPS, zstep, 0)
    zbase = sid * _ZSL
    pltpu.sync_copy(e0, accum_sh.at[pl.ds(zbase, _B)])
    pltpu.sync_copy(e0, accum_sh.at[pl.ds(zbase + _B, _B)])
    pltpu.sync_copy(e0, accum_sh.at[pl.ds(zbase + 2 * _B, _B)])
    pltpu.sync_copy(e0.at[pl.ds(0, _ZSL - 3 * _B)],
                    accum_sh.at[pl.ds(zbase + 3 * _B, _ZSL - 3 * _B)])
    plsc.subcore_barrier()

    def issue_load(t, st):
        dv, iv, jv, _, sin, _ = st
        off = (wid + t * _NW) * _B
        pltpu.async_copy(d_hbm.at[pl.ds(off, _B)], dv, sin)
        pltpu.async_copy(ii_hbm.at[pl.ds(off, _B)], iv, sin)
        pltpu.async_copy(ij_hbm.at[pl.ds(off, _B)], jv, sin)

    def wait_load(st):
        dv, iv, jv, _, sin, _ = st
        pltpu.make_async_copy(d_hbm.at[pl.ds(0, _B)], dv, sin).wait()
        pltpu.make_async_copy(ii_hbm.at[pl.ds(0, _B)], iv, sin).wait()
        pltpu.make_async_copy(ij_hbm.at[pl.ds(0, _B)], jv, sin).wait()

    def wait_scatter(st):
        _, iv, _, ev, _, ssc = st
        pltpu.make_async_copy(ev, accum_sh.at[iv], ssc).wait()

    def substep(t, k):
        st = sets[k]
        nxt = sets[(k + 1) % 3]
        @pl.when(t >= 2)
        def _():
            wait_scatter(nxt)

        @pl.when(t + 1 < _NT)
        def _():
            issue_load(t + 1, nxt)

        @pl.when(t < _NT)
        def _():
            dv, iv, jv, ev, _, ssc = st
            wait_load(st)

            def step(kk, _):
                base = kk * 64
                for u in range(4):
                    s = pl.ds(base + u * 16, 16)
                    d = dv[s]
                    qi = plsc.load_gather(table_v, [iv[s]])
                    qj = plsc.load_gather(table_v, [jv[s]])
                    uu = d * jnp.float32(_INV_H)
                    idx = jnp.minimum(uu.astype(jnp.int32), _K - 1)
                    frac = uu - idx.astype(jnp.float32)
                    mk = plsc.load_gather(mval_v, [idx])
                    sk = plsc.load_gather(mslp_v, [idx])
                    ev[s] = (qi * qj) * (mk + frac * sk)
                return 0
            lax.fori_loop(0, _STEPS // 4, step, 0)
            pltpu.async_copy(ev, accum_sh.at[iv], ssc, add=True)

    issue_load(0, sets[0])

    def trip(g, _):
        substep(3 * g, 0)
        substep(3 * g + 1, 1)
        substep(3 * g + 2, 2)
        return 0
    lax.fori_loop(0, (_NT + 3) // 3, trip, 0)

    wait_scatter(sets[(_NT - 1) % 3])

    plsc.subcore_barrier()

    def wb(base, n):
        pltpu.sync_copy(accum_sh.at[pl.ds(base, n)], e0.at[pl.ds(0, n)])

        @pl.when(cid == 0)
        def _():
            pltpu.sync_copy(e0.at[pl.ds(0, n)], out0_hbm.at[pl.ds(base, n)])

        @pl.when(cid == 1)
        def _():
            pltpu.sync_copy(e0.at[pl.ds(0, n)], out1_hbm.at[pl.ds(base, n)])
    wb(zbase, _B)
    wb(zbase + _B, _B)
    wb(zbase + 2 * _B, _B)
    wb(zbase + 3 * _B, _ZSL - 3 * _B)


@functools.partial(
    pl.kernel,
    out_type=jax.ShapeDtypeStruct((_N_PAD,), jnp.float32),
    mesh=_mesh,
    compiler_params=pltpu.CompilerParams(needs_layout_passes=False),
    scratch_types=[
        pltpu.VMEM((_CSL,), jnp.float32),
        pltpu.VMEM((_CSL,), jnp.float32),
    ],
)
def _sc_combine(p0_hbm, p1_hbm, out_hbm, a_v, b_v):
    cid = lax.axis_index("c")
    sid = lax.axis_index("s")
    wid = sid * _NC + cid
    base = wid * _CSL
    pltpu.sync_copy(p0_hbm.at[pl.ds(base, _CSL)], a_v)
    pltpu.sync_copy(p1_hbm.at[pl.ds(base, _CSL)], b_v)

    def step(k, _):
        s = pl.ds(k * 16, 16)
        a_v[s] = a_v[s] + b_v[s]
        return 0
    lax.fori_loop(0, _CSL // 16, step, 0)
    pltpu.sync_copy(a_v, out_hbm.at[pl.ds(base, _CSL)])


def kernel(atomic_charges, distances, idx_i, idx_j):
    p0, p1 = _sc_main(atomic_charges, distances, idx_i, idx_j)
    summed = _sc_combine(p0, p1)
    return summed[:_N_NODES]


# speedup vs baseline: 1.1329x; 1.1329x over previous
import functools

import jax
import jax.numpy as jnp
from jax import lax
from jax.experimental import pallas as pl
from jax.experimental.pallas import tpu as pltpu
from jax.experimental.pallas import tpu_sc as plsc

_N_NODES = 100000
_N_EDGES = 6400000
_NC = 2
_NS = 16
_NW = _NC * _NS
_B = 1600
_NBLK = _N_EDGES // _B
_NT = _NBLK // _NW
_STEPS = _B // 16
_N_PAD = 100352
_ZSL = _N_PAD // _NS
_CSL = _N_PAD // _NW

_K = 4096
_DMAX = 14.08
_H = _DMAX / _K
_INV_H = _K / _DMAX
_TABW = _K + 16

_KEHALF = 7.199822675975274
_C1 = 1.0 / 144.0
_C2 = 1.0 / 6.0
_CUTOFF = 12.0

_mesh = plsc.VectorSubcoreMesh(
    core_axis_name="c", subcore_axis_name="s", num_cores=_NC, num_subcores=_NS
)


def _m_exact(d):
    t = d * d + 1.0
    ib = plsc.bitcast(t, jnp.int32)
    ib = jnp.int32(0x5F3759DF) - lax.shift_right_arithmetic(ib, 1)
    r = plsc.bitcast(ib, jnp.float32)
    th = 0.5 * t
    r = r * (1.5 - th * r * r)
    r = r * (1.5 - th * r * r)
    r = r * (1.5 - th * r * r)
    ds_ = t * r
    e_shd = r + (ds_ * jnp.float32(_C1) - jnp.float32(_C2))
    e_ord = 1.0 / d + (d * jnp.float32(_C1) - jnp.float32(_C2))
    x = jnp.minimum(d * 0.5, 1.0)
    x2 = x * x
    x3 = x2 * x
    w_on = x3 * ((6.0 * x2 - 15.0 * x) + 10.0)
    mix = e_shd + w_on * (e_ord - e_shd)
    return jnp.where(d <= jnp.float32(_CUTOFF),
                     jnp.float32(_KEHALF) * mix, jnp.float32(0.0))


@functools.partial(
    pl.kernel,
    out_type=(jax.ShapeDtypeStruct((_N_PAD,), jnp.float32),
              jax.ShapeDtypeStruct((_N_PAD,), jnp.float32)),
    mesh=_mesh,
    compiler_params=pltpu.CompilerParams(needs_layout_passes=False),
    scratch_types=[
        pltpu.VMEM((_N_NODES,), jnp.float32),
        pltpu.VMEM((_TABW,), jnp.float32),
        pltpu.VMEM((_B,), jnp.float32),
        pltpu.VMEM((_B,), jnp.float32),
        pltpu.VMEM((_B,), jnp.float32),
        pltpu.VMEM((_B,), jnp.int32),
        pltpu.VMEM((_B,), jnp.int32),
        pltpu.VMEM((_B,), jnp.int32),
        pltpu.VMEM((_B,), jnp.int32),
        pltpu.VMEM((_B,), jnp.int32),
        pltpu.VMEM((_B,), jnp.int32),
        pltpu.VMEM((_B,), jnp.float32),
        pltpu.VMEM((_B,), jnp.float32),
        pltpu.VMEM((_B,), jnp.float32),
        pltpu.VMEM_SHARED((_N_PAD,), jnp.float32),
        pltpu.SemaphoreType.DMA,
        pltpu.SemaphoreType.DMA,
        pltpu.SemaphoreType.DMA,
        pltpu.SemaphoreType.DMA,
        pltpu.SemaphoreType.DMA,
        pltpu.SemaphoreType.DMA,
    ],
)
def _sc_main(ch_hbm, d_hbm, ii_hbm, ij_hbm, out0_hbm, out1_hbm,
             table_v, mval_v,
             d0, d1, d2, i0, i1, i2, j0, j1, j2, e0, e1, e2,
             accum_sh, si0, si1, si2, ss0, ss1, ss2):
    cid = lax.axis_index("c")
    sid = lax.axis_index("s")
    wid = sid * _NC + cid
    sets = ((d0, i0, j0, e0, si0, ss0),
            (d1, i1, j1, e1, si1, ss1),
            (d2, i2, j2, e2, si2, ss2))

    pltpu.sync_copy(ch_hbm, table_v)

    lanes = lax.iota(jnp.int32, 16)

    def build_vals(k, _):
        node = k * 16 + lanes
        dn = (node.astype(jnp.float32) + 0.5) * jnp.float32(_H)
        mval_v[pl.ds(k * 16, 16)] = _m_exact(dn)
        return 0
    lax.fori_loop(0, _K // 16, build_vals, 0)

    def zstep(k, _):
        e0[pl.ds(k * 16, 16)] = jnp.zeros((16,), jnp.float32)
        return 0
    lax.fori_loop(0, _STEPS, zstep, 0)
    zbase = sid * _ZSL
    pltpu.sync_copy(e0, accum_sh.at[pl.ds(zbase, _B)])
    pltpu.sync_copy(e0, accum_sh.at[pl.ds(zbase + _B, _B)])
    pltpu.sync_copy(e0, accum_sh.at[pl.ds(zbase + 2 * _B, _B)])
    pltpu.sync_copy(e0.at[pl.ds(0, _ZSL - 3 * _B)],
                    accum_sh.at[pl.ds(zbase + 3 * _B, _ZSL - 3 * _B)])
    plsc.subcore_barrier()

    def issue_load(t, st):
        dv, iv, jv, _, sin, _ = st
        off = (wid + t * _NW) * _B
        pltpu.async_copy(d_hbm.at[pl.ds(off, _B)], dv, sin)
        pltpu.async_copy(ii_hbm.at[pl.ds(off, _B)], iv, sin)
        pltpu.async_copy(ij_hbm.at[pl.ds(off, _B)], jv, sin)

    def wait_load(st):
        dv, iv, jv, _, sin, _ = st
        pltpu.make_async_copy(d_hbm.at[pl.ds(0, _B)], dv, sin).wait()
        pltpu.make_async_copy(ii_hbm.at[pl.ds(0, _B)], iv, sin).wait()
        pltpu.make_async_copy(ij_hbm.at[pl.ds(0, _B)], jv, sin).wait()

    def wait_scatter(st):
        _, iv, _, ev, _, ssc = st
        pltpu.make_async_copy(ev, accum_sh.at[iv], ssc).wait()

    def substep(t, k):
        st = sets[k]
        nxt = sets[(k + 1) % 3]
        @pl.when(t >= 2)
        def _():
            wait_scatter(nxt)

        @pl.when(t + 1 < _NT)
        def _():
            issue_load(t + 1, nxt)

        @pl.when(t < _NT)
        def _():
            dv, iv, jv, ev, _, ssc = st
            wait_load(st)

            def step(kk, _):
                base = kk * 64
                for u in range(4):
                    s = pl.ds(base + u * 16, 16)
                    d = dv[s]
                    qi = plsc.load_gather(table_v, [iv[s]])
                    qj = plsc.load_gather(table_v, [jv[s]])
                    uu = d * jnp.float32(_INV_H)
                    idx = jnp.minimum(uu.astype(jnp.int32), _K - 1)
                    mk = plsc.load_gather(mval_v, [idx])
                    ev[s] = (qi * qj) * mk
                return 0
            lax.fori_loop(0, _STEPS // 4, step, 0)
            pltpu.async_copy(ev, accum_sh.at[iv], ssc, add=True)

    issue_load(0, sets[0])

    def trip(g, _):
        substep(3 * g, 0)
        substep(3 * g + 1, 1)
        substep(3 * g + 2, 2)
        return 0
    lax.fori_loop(0, (_NT + 3) // 3, trip, 0)

    wait_scatter(sets[(_NT - 1) % 3])

    plsc.subcore_barrier()

    def wb(base, n):
        pltpu.sync_copy(accum_sh.at[pl.ds(base, n)], e0.at[pl.ds(0, n)])

        @pl.when(cid == 0)
        def _():
            pltpu.sync_copy(e0.at[pl.ds(0, n)], out0_hbm.at[pl.ds(base, n)])

        @pl.when(cid == 1)
        def _():
            pltpu.sync_copy(e0.at[pl.ds(0, n)], out1_hbm.at[pl.ds(base, n)])
    wb(zbase, _B)
    wb(zbase + _B, _B)
    wb(zbase + 2 * _B, _B)
    wb(zbase + 3 * _B, _ZSL - 3 * _B)


@functools.partial(
    pl.kernel,
    out_type=jax.ShapeDtypeStruct((_N_PAD,), jnp.float32),
    mesh=_mesh,
    compiler_params=pltpu.CompilerParams(needs_layout_passes=False),
    scratch_types=[
        pltpu.VMEM((_CSL,), jnp.float32),
        pltpu.VMEM((_CSL,), jnp.float32),
    ],
)
def _sc_combine(p0_hbm, p1_hbm, out_hbm, a_v, b_v):
    cid = lax.axis_index("c")
    sid = lax.axis_index("s")
    wid = sid * _NC + cid
    base = wid * _CSL
    pltpu.sync_copy(p0_hbm.at[pl.ds(base, _CSL)], a_v)
    pltpu.sync_copy(p1_hbm.at[pl.ds(base, _CSL)], b_v)

    def step(k, _):
        s = pl.ds(k * 16, 16)
        a_v[s] = a_v[s] + b_v[s]
        return 0
    lax.fori_loop(0, _CSL // 16, step, 0)
    pltpu.sync_copy(a_v, out_hbm.at[pl.ds(base, _CSL)])


def kernel(atomic_charges, distances, idx_i, idx_j):
    p0, p1 = _sc_main(atomic_charges, distances, idx_i, idx_j)
    summed = _sc_combine(p0, p1)
    return summed[:_N_NODES]


# speedup vs baseline: 1.1350x; 1.0018x over previous
import functools

import jax
import jax.numpy as jnp
from jax import lax
from jax.experimental import pallas as pl
from jax.experimental.pallas import tpu as pltpu
from jax.experimental.pallas import tpu_sc as plsc

_N_NODES = 100000
_N_EDGES = 6400000
_NC = 2
_NS = 16
_NW = _NC * _NS
_B = 1600
_NBLK = _N_EDGES // _B
_NT = _NBLK // _NW
_STEPS = _B // 16
_N_PAD = 100352
_ZSL = _N_PAD // _NS
_CSL = _N_PAD // _NW

_K = 4096
_DMAX = 14.08
_H = _DMAX / _K
_INV_H = _K / _DMAX
_TABW = _K + 16

_KEHALF = 7.199822675975274
_C1 = 1.0 / 144.0
_C2 = 1.0 / 6.0
_CUTOFF = 12.0

_mesh = plsc.VectorSubcoreMesh(
    core_axis_name="c", subcore_axis_name="s", num_cores=_NC, num_subcores=_NS
)


def _m_exact(d):
    t = d * d + 1.0
    ib = plsc.bitcast(t, jnp.int32)
    ib = jnp.int32(0x5F3759DF) - lax.shift_right_arithmetic(ib, 1)
    r = plsc.bitcast(ib, jnp.float32)
    th = 0.5 * t
    r = r * (1.5 - th * r * r)
    r = r * (1.5 - th * r * r)
    r = r * (1.5 - th * r * r)
    ds_ = t * r
    e_shd = r + (ds_ * jnp.float32(_C1) - jnp.float32(_C2))
    e_ord = 1.0 / d + (d * jnp.float32(_C1) - jnp.float32(_C2))
    x = jnp.minimum(d * 0.5, 1.0)
    x2 = x * x
    x3 = x2 * x
    w_on = x3 * ((6.0 * x2 - 15.0 * x) + 10.0)
    mix = e_shd + w_on * (e_ord - e_shd)
    return jnp.where(d <= jnp.float32(_CUTOFF),
                     jnp.float32(_KEHALF) * mix, jnp.float32(0.0))


@functools.partial(
    pl.kernel,
    out_type=(jax.ShapeDtypeStruct((_N_PAD,), jnp.float32),
              jax.ShapeDtypeStruct((_N_PAD,), jnp.float32)),
    mesh=_mesh,
    compiler_params=pltpu.CompilerParams(needs_layout_passes=False),
    scratch_types=[
        pltpu.VMEM((_N_NODES,), jnp.float32),
        pltpu.VMEM((_TABW,), jnp.float32),
        pltpu.VMEM((_B,), jnp.float32),
        pltpu.VMEM((_B,), jnp.float32),
        pltpu.VMEM((_B,), jnp.float32),
        pltpu.VMEM((_B,), jnp.int32),
        pltpu.VMEM((_B,), jnp.int32),
        pltpu.VMEM((_B,), jnp.int32),
        pltpu.VMEM((_B,), jnp.int32),
        pltpu.VMEM((_B,), jnp.int32),
        pltpu.VMEM((_B,), jnp.int32),
        pltpu.VMEM((_B,), jnp.float32),
        pltpu.VMEM((_B,), jnp.float32),
        pltpu.VMEM((_B,), jnp.float32),
        pltpu.VMEM_SHARED((_N_PAD,), jnp.float32),
        pltpu.SemaphoreType.DMA,
        pltpu.SemaphoreType.DMA,
        pltpu.SemaphoreType.DMA,
        pltpu.SemaphoreType.DMA,
        pltpu.SemaphoreType.DMA,
        pltpu.SemaphoreType.DMA,
    ],
)
def _sc_main(ch_hbm, d_hbm, ii_hbm, ij_hbm, out0_hbm, out1_hbm,
             table_v, mval_v,
             d0, d1, d2, i0, i1, i2, j0, j1, j2, e0, e1, e2,
             accum_sh, si0, si1, si2, ss0, ss1, ss2):
    cid = lax.axis_index("c")
    sid = lax.axis_index("s")
    wid = sid * _NC + cid
    sets = ((d0, i0, j0, e0, si0, ss0),
            (d1, i1, j1, e1, si1, ss1),
            (d2, i2, j2, e2, si2, ss2))

    def issue_load(t, st):
        dv, iv, jv, _, sin, _ = st
        off = (wid + t * _NW) * _B
        pltpu.async_copy(d_hbm.at[pl.ds(off, _B)], dv, sin)
        pltpu.async_copy(ii_hbm.at[pl.ds(off, _B)], iv, sin)
        pltpu.async_copy(ij_hbm.at[pl.ds(off, _B)], jv, sin)

    def wait_load(st):
        dv, iv, jv, _, sin, _ = st
        pltpu.make_async_copy(d_hbm.at[pl.ds(0, _B)], dv, sin).wait()
        pltpu.make_async_copy(ii_hbm.at[pl.ds(0, _B)], iv, sin).wait()
        pltpu.make_async_copy(ij_hbm.at[pl.ds(0, _B)], jv, sin).wait()

    def wait_scatter(st):
        _, iv, _, ev, _, ssc = st
        pltpu.make_async_copy(ev, accum_sh.at[iv], ssc).wait()

    issue_load(0, sets[0])
    issue_load(1, sets[1])

    pltpu.sync_copy(ch_hbm, table_v)

    lanes = lax.iota(jnp.int32, 16)

    def build_vals(k, _):
        node = k * 16 + lanes
        dn = (node.astype(jnp.float32) + 0.5) * jnp.float32(_H)
        mval_v[pl.ds(k * 16, 16)] = _m_exact(dn)
        return 0
    lax.fori_loop(0, _K // 16, build_vals, 0)

    def zstep(k, _):
        e0[pl.ds(k * 16, 16)] = jnp.zeros((16,), jnp.float32)
        return 0
    lax.fori_loop(0, _STEPS, zstep, 0)
    zbase = sid * _ZSL
    pltpu.sync_copy(e0, accum_sh.at[pl.ds(zbase, _B)])
    pltpu.sync_copy(e0, accum_sh.at[pl.ds(zbase + _B, _B)])
    pltpu.sync_copy(e0, accum_sh.at[pl.ds(zbase + 2 * _B, _B)])
    pltpu.sync_copy(e0.at[pl.ds(0, _ZSL - 3 * _B)],
                    accum_sh.at[pl.ds(zbase + 3 * _B, _ZSL - 3 * _B)])
    plsc.subcore_barrier()

    def substep(t, k):
        st = sets[k]
        nxt = sets[(k + 1) % 3]
        @pl.when(t >= 2)
        def _():
            wait_scatter(nxt)

        @pl.when(jnp.logical_and(t + 1 < _NT, t >= 1))
        def _():
            issue_load(t + 1, nxt)

        @pl.when(t < _NT)
        def _():
            dv, iv, jv, ev, _, ssc = st
            wait_load(st)

            def step(kk, _):
                base = kk * 64
                for u in range(4):
                    s = pl.ds(base + u * 16, 16)
                    d = dv[s]
                    qi = plsc.load_gather(table_v, [iv[s]])
                    qj = plsc.load_gather(table_v, [jv[s]])
                    uu = d * jnp.float32(_INV_H)
                    idx = jnp.minimum(uu.astype(jnp.int32), _K - 1)
                    mk = plsc.load_gather(mval_v, [idx])
                    ev[s] = (qi * qj) * mk
                return 0
            lax.fori_loop(0, _STEPS // 4, step, 0)
            pltpu.async_copy(ev, accum_sh.at[iv], ssc, add=True)

    def trip(g, _):
        substep(3 * g, 0)
        substep(3 * g + 1, 1)
        substep(3 * g + 2, 2)
        return 0
    lax.fori_loop(0, (_NT + 3) // 3, trip, 0)

    wait_scatter(sets[(_NT - 1) % 3])

    plsc.subcore_barrier()

    def wb(base, n):
        pltpu.sync_copy(accum_sh.at[pl.ds(base, n)], e0.at[pl.ds(0, n)])

        @pl.when(cid == 0)
        def _():
            pltpu.sync_copy(e0.at[pl.ds(0, n)], out0_hbm.at[pl.ds(base, n)])

        @pl.when(cid == 1)
        def _():
            pltpu.sync_copy(e0.at[pl.ds(0, n)], out1_hbm.at[pl.ds(base, n)])
    wb(zbase, _B)
    wb(zbase + _B, _B)
    wb(zbase + 2 * _B, _B)
    wb(zbase + 3 * _B, _ZSL - 3 * _B)


@functools.partial(
    pl.kernel,
    out_type=jax.ShapeDtypeStruct((_N_PAD,), jnp.float32),
    mesh=_mesh,
    compiler_params=pltpu.CompilerParams(needs_layout_passes=False),
    scratch_types=[
        pltpu.VMEM((_CSL,), jnp.float32),
        pltpu.VMEM((_CSL,), jnp.float32),
    ],
)
def _sc_combine(p0_hbm, p1_hbm, out_hbm, a_v, b_v):
    cid = lax.axis_index("c")
    sid = lax.axis_index("s")
    wid = sid * _NC + cid
    base = wid * _CSL
    pltpu.sync_copy(p0_hbm.at[pl.ds(base, _CSL)], a_v)
    pltpu.sync_copy(p1_hbm.at[pl.ds(base, _CSL)], b_v)

    def step(k, _):
        s = pl.ds(k * 16, 16)
        a_v[s] = a_v[s] + b_v[s]
        return 0
    lax.fori_loop(0, _CSL // 16, step, 0)
    pltpu.sync_copy(a_v, out_hbm.at[pl.ds(base, _CSL)])


def kernel(atomic_charges, distances, idx_i, idx_j):
    p0, p1 = _sc_main(atomic_charges, distances, idx_i, idx_j)
    summed = _sc_combine(p0, p1)
    return summed[:_N_NODES]
